# baseline (speedup 1.0000x reference)
import functools
import jax
import jax.numpy as jnp
from jax.experimental import pallas as pl
from jax.experimental.pallas import tpu as pltpu


def _ceil_to(a, m):
    return (a + m - 1) // m * m


def _pad2(a, m0, m1):
    s0, s1 = a.shape
    p0, p1 = _ceil_to(s0, m0) - s0, _ceil_to(s1, m1) - s1
    if p0 or p1:
        a = jnp.pad(a, ((0, p0), (0, p1)))
    return a


def _mm_kernel(x_ref, w_ref, b_ref, o_ref, acc_ref, *, nk, act, fuse2):
    k = pl.program_id(2)

    @pl.when(k == 0)
    def _():
        acc_ref[...] = jnp.zeros_like(acc_ref)

    if fuse2:
        xb = jnp.maximum(x_ref[0] + x_ref[1], 0.0)
    else:
        xb = x_ref[...]
    acc_ref[...] += jnp.dot(xb, w_ref[...],
                            preferred_element_type=jnp.float32)

    @pl.when(k == nk - 1)
    def _():
        y = acc_ref[...] + b_ref[...]
        if act == "relu":
            y = jnp.maximum(y, 0.0)
        elif act == "tanh":
            y = jnp.tanh(y)
        o_ref[...] = y


def _dense(x, w, b, act="relu", x2=None):
    M, K = x.shape
    N = w.shape[1]
    BM, BN, BK = 512, 512, 512
    xp = _pad2(x, BM, BK)
    if x2 is not None:
        xp = jnp.stack([xp, _pad2(x2, BM, BK)])
    wp = _pad2(w, BK, BN)
    Mp, Kp = xp.shape[-2:]
    Np = wp.shape[1]
    bn = min(BN, Np)
    bm = min(BM, Mp)
    bk = min(BK, Kp)
    bp = jnp.pad(b, (0, Np - N)).reshape(1, Np)
    nk = Kp // bk
    if x2 is None:
        x_spec = pl.BlockSpec((bm, bk), lambda i, j, k: (i, k))
    else:
        x_spec = pl.BlockSpec((2, bm, bk), lambda i, j, k: (0, i, k))
    out = pl.pallas_call(
        functools.partial(_mm_kernel, nk=nk, act=act, fuse2=x2 is not None),
        grid=(Mp // bm, Np // bn, nk),
        in_specs=[
            x_spec,
            pl.BlockSpec((bk, bn), lambda i, j, k: (k, j)),
            pl.BlockSpec((1, bn), lambda i, j, k: (0, j)),
        ],
        out_specs=pl.BlockSpec((bm, bn), lambda i, j, k: (i, j)),
        out_shape=jax.ShapeDtypeStruct((Mp, Np), jnp.float32),
        scratch_shapes=[pltpu.VMEM((bm, bn), jnp.float32)],
        compiler_params=pltpu.CompilerParams(
            dimension_semantics=("parallel", "parallel", "arbitrary")),
    )(xp, wp, bp)
    return out[:M, :N]


def _pool_kernel(x_ref, ids_ref, o_ref, *, nb, nseg):
    i = pl.program_id(0)

    @pl.when(i == 0)
    def _():
        o_ref[...] = jnp.zeros_like(o_ref)

    x = x_ref[...]
    ids = ids_ref[...]
    for b in range(nseg):
        red = jnp.max(jnp.where(ids == b, x, 0.0), axis=0, keepdims=True)
        o_ref[b, :] = jnp.maximum(o_ref[b, :], red[0])


def _batch_pool(x, batch, nseg):
    M, C = x.shape
    BM = 512
    xp = _pad2(x, BM, 128)
    Mp, Cp = xp.shape
    idp = jnp.pad(batch.astype(jnp.int32), (0, Mp - M),
                  constant_values=nseg)
    nb = Mp // BM
    idp = idp.reshape(Mp, 1)
    out = pl.pallas_call(
        functools.partial(_pool_kernel, nb=nb, nseg=nseg),
        grid=(nb,),
        in_specs=[
            pl.BlockSpec((BM, Cp), lambda i: (i, 0)),
            pl.BlockSpec((BM, 1), lambda i: (i, 0)),
        ],
        out_specs=pl.BlockSpec((nseg, Cp), lambda i: (0, 0)),
        out_shape=jax.ShapeDtypeStruct((nseg, Cp), jnp.float32),
        compiler_params=pltpu.CompilerParams(
            dimension_semantics=("arbitrary",)),
    )(xp, idp)
    return out[:, :C]


def _gather_rows(table, idx):
    return jnp.take(table, idx, axis=0)


def _scatter_max(msg, dst, n):
    out = jax.ops.segment_max(msg, dst, num_segments=n)
    return jnp.maximum(out, 0.0)


def _edge_conv(x, src, dst, layers, n):
    (w1, b1), (w2, b2) = layers
    h = w1.shape[0]
    c = x.shape[1]
    A, Bm = w1[:, :c], w1[:, c:]
    uv = _dense(x, jnp.concatenate([(A - Bm).T, Bm.T], axis=1),
                jnp.concatenate([b1, jnp.zeros_like(b1)]), act="none")
    u, v = uv[:, :h], uv[:, h:]
    g1 = _gather_rows(u, dst)
    g2 = _gather_rows(v, src)
    h2 = _dense(g1, w2.T, b2, act="relu", x2=g2)
    return _scatter_max(h2, dst, n)


def _gcu(x, tpl_src, tpl_dst, geo_src, geo_dst, tpl_p, geo_p, mlp_p, n):
    xt = _edge_conv(x, tpl_src, tpl_dst, tpl_p, n)
    xg = _edge_conv(x, geo_src, geo_dst, geo_p, n)
    (wm, bm), = mlp_p
    return _dense(jnp.concatenate([xt, xg], axis=1), wm.T, bm, act="relu")


def kernel(pos, x, tpl_edge_index, geo_edge_index, batch, g1_tpl, g1_geo,
           g1_mlp, g2_tpl, g2_geo, g2_mlp, g3_tpl, g3_geo, g3_mlp, glb,
           trans_mlp, trans_w, trans_b):
    n = pos.shape[0]
    nseg = 8
    ts, td = tpl_edge_index[0], tpl_edge_index[1]
    gs, gd = geo_edge_index[0], geo_edge_index[1]
    xin = jnp.concatenate([pos, x], axis=1)
    x1 = _gcu(xin, ts, td, gs, gd, g1_tpl, g1_geo, g1_mlp, n)
    x2 = _gcu(x1, ts, td, gs, gd, g2_tpl, g2_geo, g2_mlp, n)
    x3 = _gcu(x2, ts, td, gs, gd, g3_tpl, g3_geo, g3_mlp, n)
    (wg, bg), = glb
    x4 = _dense(jnp.concatenate([x1, x2, x3], axis=1), wg.T, bg, act="relu")
    xg = _batch_pool(x4, batch, nseg)
    onehot = (batch[:, None] == jnp.arange(nseg)[None, :]).astype(jnp.float32)
    xgn = _dense(onehot, xg, jnp.zeros((xg.shape[1],), jnp.float32),
                 act="none")
    x5 = jnp.concatenate([xgn, xin, x1, x2, x3], axis=1)
    (wt1, bt1), (wt2, bt2) = trans_mlp
    h = _dense(x5, wt1.T, bt1, act="relu")
    h = _dense(h, wt2.T, bt2, act="relu")
    return _dense(h, trans_w.T, trans_b, act="tanh")
